# initial kernel scaffold (unmeasured)
import jax
import jax.numpy as jnp
from jax import lax
from jax.experimental import pallas as pl
from jax.experimental.pallas import tpu as pltpu

N_DEV = 32


def kernel(x, w_mat):
    m_per, k = x.shape
    _, n = w_mat.shape
    n_per = n // N_DEV
    m = m_per * N_DEV

    def body(x_ref, w_hbm, out_ref, w_buf, y_buf, copy_sem, send_sem, recv_sems):
        my = lax.axis_index("i")

        def compute_block(s):
            j = lax.rem(my + s, N_DEV)
            cp = pltpu.make_async_copy(
                w_hbm.at[:, pl.ds(j * n_per, n_per)], w_buf, copy_sem
            )
            cp.start()
            cp.wait()
            y = jnp.dot(
                x_ref[...],
                w_buf[...],
                preferred_element_type=jnp.float32,
                precision=lax.Precision.HIGHEST,
            )
            return y * jax.nn.sigmoid(y)

        out_ref[pl.ds(my * m_per, m_per), :] = compute_block(0)

        for s in range(1, N_DEV):
            j = lax.rem(my + s, N_DEV)
            y_buf[...] = compute_block(s)
            rdma = pltpu.make_async_remote_copy(
                src_ref=y_buf,
                dst_ref=out_ref.at[pl.ds(my * m_per, m_per), :],
                send_sem=send_sem,
                recv_sem=recv_sems.at[s],
                device_id=(j,),
                device_id_type=pl.DeviceIdType.MESH,
            )
            rdma.start()
            rdma.wait_send()

        for s in range(1, N_DEV):
            src = lax.rem(my - s + N_DEV, N_DEV)
            recv = pltpu.make_async_remote_copy(
                src_ref=y_buf,
                dst_ref=out_ref.at[pl.ds(src * m_per, m_per), :],
                send_sem=send_sem,
                recv_sem=recv_sems.at[s],
                device_id=(src,),
                device_id_type=pl.DeviceIdType.MESH,
            )
            recv.wait_recv()

    return pl.pallas_call(
        body,
        out_shape=jax.ShapeDtypeStruct((m, n_per), jnp.float32),
        in_specs=[
            pl.BlockSpec(memory_space=pltpu.VMEM),
            pl.BlockSpec(memory_space=pltpu.ANY),
        ],
        out_specs=pl.BlockSpec(memory_space=pltpu.VMEM),
        scratch_shapes=[
            pltpu.VMEM((k, n_per), jnp.float32),
            pltpu.VMEM((m_per, n_per), jnp.float32),
            pltpu.SemaphoreType.DMA,
            pltpu.SemaphoreType.DMA,
            pltpu.SemaphoreType.DMA((N_DEV,)),
        ],
        compiler_params=pltpu.CompilerParams(collective_id=0),
    )(x, w_mat)


# baseline (device time: 268027 ns/iter reference)
import jax
import jax.numpy as jnp
from jax import lax
from jax.experimental import pallas as pl
from jax.experimental.pallas import tpu as pltpu

N_DEV = 32


def kernel(x, w_mat):
    m_per, k = x.shape
    _, n = w_mat.shape
    n_per = n // N_DEV
    m = m_per * N_DEV

    def body(x_ref, w_hbm, out_ref, w_buf, y_buf, copy_sem, send_sem, recv_sems):
        my = lax.axis_index("i")

        def compute_block(s):
            j = lax.rem(my + s, N_DEV)
            cp = pltpu.make_async_copy(
                w_hbm.at[:, pl.ds(j * n_per, n_per)], w_buf, copy_sem
            )
            cp.start()
            cp.wait()
            y = jnp.dot(
                x_ref[...],
                w_buf[...],
                preferred_element_type=jnp.float32,
                precision=lax.Precision.HIGHEST,
            )
            return y * jax.nn.sigmoid(y)

        out_ref[pl.ds(my * m_per, m_per), :] = compute_block(0)

        for s in range(1, N_DEV):
            j = lax.rem(my + s, N_DEV)
            y_buf[...] = compute_block(s)
            rdma = pltpu.make_async_remote_copy(
                src_ref=y_buf,
                dst_ref=out_ref.at[pl.ds(my * m_per, m_per), :],
                send_sem=send_sem,
                recv_sem=recv_sems.at[s],
                device_id=(j,),
                device_id_type=pl.DeviceIdType.MESH,
            )
            rdma.start()
            rdma.wait_send()

        for s in range(1, N_DEV):
            src = lax.rem(my - s + N_DEV, N_DEV)
            recv = pltpu.make_async_remote_copy(
                src_ref=y_buf,
                dst_ref=out_ref.at[pl.ds(src * m_per, m_per), :],
                send_sem=send_sem,
                recv_sem=recv_sems.at[s],
                device_id=(src,),
                device_id_type=pl.DeviceIdType.MESH,
            )
            recv.wait_recv()

    return pl.pallas_call(
        body,
        out_shape=jax.ShapeDtypeStruct((m, n_per), jnp.float32),
        in_specs=[
            pl.BlockSpec(memory_space=pltpu.VMEM),
            pl.BlockSpec(memory_space=pl.ANY),
        ],
        out_specs=pl.BlockSpec(memory_space=pltpu.VMEM),
        scratch_shapes=[
            pltpu.VMEM((k, n_per), jnp.float32),
            pltpu.VMEM((m_per, n_per), jnp.float32),
            pltpu.SemaphoreType.DMA,
            pltpu.SemaphoreType.DMA,
            pltpu.SemaphoreType.DMA((N_DEV,)),
        ],
    )(x, w_mat)


# device time: 135856 ns/iter; 1.9729x vs baseline; 1.9729x over previous
import jax
import jax.numpy as jnp
from jax import lax
from jax.experimental import pallas as pl
from jax.experimental.pallas import tpu as pltpu

N_DEV = 32
N_SLOTS = 4


def kernel(x, w_mat):
    m_per, k = x.shape
    _, n = w_mat.shape
    n_per = n // N_DEV
    m = m_per * N_DEV

    def body(x_ref, w_hbm, out_ref, w_buf, y_buf, copy_sems, send_sems, recv_sems):
        my = lax.axis_index("i")

        def w_copy(s, slot):
            j = lax.rem(my + s, N_DEV)
            return pltpu.make_async_copy(
                w_hbm.at[:, pl.ds(j * n_per, n_per)],
                w_buf.at[slot],
                copy_sems.at[slot],
            )

        def send_desc(s):
            slot = s % N_SLOTS
            j = lax.rem(my + s, N_DEV)
            return pltpu.make_async_remote_copy(
                src_ref=y_buf.at[slot],
                dst_ref=out_ref.at[pl.ds(my * m_per, m_per), :],
                send_sem=send_sems.at[slot],
                recv_sem=recv_sems.at[s],
                device_id=(j,),
                device_id_type=pl.DeviceIdType.MESH,
            )

        w_copy(0, 0).start()
        for s in range(N_DEV):
            w_slot = s % 2
            if s + 1 < N_DEV:
                w_copy(s + 1, 1 - w_slot).start()
            w_copy(s, w_slot).wait()
            y = jnp.dot(
                x_ref[...],
                w_buf[w_slot],
                preferred_element_type=jnp.float32,
                precision=lax.Precision.HIGHEST,
            )
            y = y * jax.nn.sigmoid(y)
            if s == 0:
                out_ref[pl.ds(my * m_per, m_per), :] = y
            else:
                if s - N_SLOTS >= 1:
                    send_desc(s - N_SLOTS).wait_send()
                y_buf[s % N_SLOTS] = y
                send_desc(s).start()

        for s in range(max(1, N_DEV - N_SLOTS), N_DEV):
            send_desc(s).wait_send()

        for s in range(1, N_DEV):
            src = lax.rem(my - s + N_DEV, N_DEV)
            recv = pltpu.make_async_remote_copy(
                src_ref=y_buf.at[0],
                dst_ref=out_ref.at[pl.ds(src * m_per, m_per), :],
                send_sem=send_sems.at[0],
                recv_sem=recv_sems.at[s],
                device_id=(src,),
                device_id_type=pl.DeviceIdType.MESH,
            )
            recv.wait_recv()

    return pl.pallas_call(
        body,
        out_shape=jax.ShapeDtypeStruct((m, n_per), jnp.float32),
        in_specs=[
            pl.BlockSpec(memory_space=pltpu.VMEM),
            pl.BlockSpec(memory_space=pl.ANY),
        ],
        out_specs=pl.BlockSpec(memory_space=pltpu.VMEM),
        scratch_shapes=[
            pltpu.VMEM((2, k, n_per), jnp.float32),
            pltpu.VMEM((N_SLOTS, m_per, n_per), jnp.float32),
            pltpu.SemaphoreType.DMA((2,)),
            pltpu.SemaphoreType.DMA((N_SLOTS,)),
            pltpu.SemaphoreType.DMA((N_DEV,)),
        ],
    )(x, w_mat)
